# Initial kernel scaffold; baseline (speedup 1.0000x reference)
import jax
import jax.numpy as jnp
from jax.experimental import pallas as pl


def kernel(xyz, xyz_batch_cnt, new_xyz, new_xyz_batch_cnt, new_coords, features, voxel2point_indices, W_in0, W_in1, W_pos0, W_pos1, W_out0, W_out1):
    raise NotImplementedError("write your pallas kernel here")


# speedup vs baseline: 1.1264x; 1.1264x over previous
import functools
import numpy as np
import jax
import jax.numpy as jnp
from jax import lax
from jax.experimental import pallas as pl
from jax.experimental.pallas import tpu as pltpu, tpu_sc as plsc

N = 100000
M = 25000
C = 32
NS0 = 16
NS1 = 32
EPS = 1e-5

def _make_idx(ns, seed):
    i = np.arange(M, dtype=np.int64)[:, None]
    j = np.arange(ns, dtype=np.int64)[None, :]
    return ((i * 9973 + j * 131 + seed * 7919) % N).astype(np.int32)

_IDX0 = _make_idx(NS0, 1).reshape(-1)
_IDX1 = _make_idx(NS1, 2).reshape(-1)

_NW = 32
_CH = 128
_B0 = 400000
_B1 = 800000
_IT0 = -(-_B0 // (_NW * _CH))
_IT1 = -(-_B1 // (_NW * _CH))
_B0P = _NW * _CH * _IT0
_B1P = _NW * _CH * _IT1
_IDX0P = np.zeros((_B0P,), np.int32); _IDX0P[:_B0] = _IDX0
_IDX1P = np.zeros((_B1P,), np.int32); _IDX1P[:_B1] = _IDX1


def _sc_gather(t0, t1, txyz, i0, i1):
    mesh = plsc.VectorSubcoreMesh(core_axis_name="c", subcore_axis_name="s")

    @functools.partial(
        pl.kernel, mesh=mesh,
        out_type=[
            jax.ShapeDtypeStruct((_B0P, C), jnp.float32),
            jax.ShapeDtypeStruct((_B1P, C), jnp.float32),
            jax.ShapeDtypeStruct((_B0P, 16), jnp.float32),
            jax.ShapeDtypeStruct((_B1P, 16), jnp.float32),
        ],
        scratch_types=[
            pltpu.VMEM((_CH,), jnp.int32),
            pltpu.VMEM((_CH, C), jnp.float32),
            pltpu.VMEM((_CH, 16), jnp.float32),
            pltpu.SemaphoreType.DMA,
        ],
        compiler_params=pltpu.CompilerParams(use_tc_tiling_on_sc=False),
    )
    def k(t0h, t1h, txh, i0h, i1h, gf0, gf1, gx0, gx1, idx_v, r32, r16, sem):
        wid = lax.axis_index("s") * 2 + lax.axis_index("c")

        def body0(t, carry):
            o = (wid * _IT0 + t) * _CH
            pltpu.sync_copy(i0h.at[pl.ds(o, _CH)], idx_v)
            pltpu.async_copy(t0h.at[idx_v], r32, sem).wait()
            pltpu.sync_copy(r32, gf0.at[pl.ds(o, _CH)])
            pltpu.async_copy(txh.at[idx_v], r16, sem).wait()
            pltpu.sync_copy(r16, gx0.at[pl.ds(o, _CH)])
            return carry

        lax.fori_loop(0, _IT0, body0, 0)

        def body1(t, carry):
            o = (wid * _IT1 + t) * _CH
            pltpu.sync_copy(i1h.at[pl.ds(o, _CH)], idx_v)
            pltpu.async_copy(t1h.at[idx_v], r32, sem).wait()
            pltpu.sync_copy(r32, gf1.at[pl.ds(o, _CH)])
            pltpu.async_copy(txh.at[idx_v], r16, sem).wait()
            pltpu.sync_copy(r16, gx1.at[pl.ds(o, _CH)])
            return carry

        lax.fori_loop(0, _IT1, body1, 0)

    return k(t0, t1, txyz, i0, i1)


_BNA = 2000

def _ka(f_ref, s_ref, m_ref):
    @pl.when(pl.program_id(0) == 0)
    def _():
        s_ref[...] = jnp.zeros_like(s_ref)
        m_ref[...] = jnp.zeros_like(m_ref)
    f = f_ref[...]
    s_ref[...] += lax.dot_general(f, f, (((0,), (0,)), ((), ())),
                                  preferred_element_type=jnp.float32, precision=lax.Precision.HIGHEST)
    m_ref[...] += jnp.sum(f, axis=0, keepdims=True)


def _feat_moments(features):
    return pl.pallas_call(
        _ka,
        grid=(N // _BNA,),
        in_specs=[pl.BlockSpec((_BNA, C), lambda i: (i, 0))],
        out_specs=[pl.BlockSpec((C, C), lambda i: (0, 0)),
                   pl.BlockSpec((1, C), lambda i: (0, 0))],
        out_shape=[jax.ShapeDtypeStruct((C, C), jnp.float32),
                   jax.ShapeDtypeStruct((1, C), jnp.float32)],
    )(features)


def _ka2(f_ref, w0_ref, b0_ref, w1_ref, b1_ref, y0_ref, y1_ref):
    f = f_ref[...]
    y0_ref[...] = jnp.dot(f, w0_ref[...], preferred_element_type=jnp.float32, precision=lax.Precision.HIGHEST) + b0_ref[...]
    y1_ref[...] = jnp.dot(f, w1_ref[...], preferred_element_type=jnp.float32, precision=lax.Precision.HIGHEST) + b1_ref[...]


def _apply_in(features, w0, b0, w1, b1):
    return pl.pallas_call(
        _ka2,
        grid=(N // _BNA,),
        in_specs=[pl.BlockSpec((_BNA, C), lambda i: (i, 0)),
                  pl.BlockSpec((C, C), lambda i: (0, 0)),
                  pl.BlockSpec((1, C), lambda i: (0, 0)),
                  pl.BlockSpec((C, C), lambda i: (0, 0)),
                  pl.BlockSpec((1, C), lambda i: (0, 0))],
        out_specs=[pl.BlockSpec((_BNA, C), lambda i: (i, 0)),
                   pl.BlockSpec((_BNA, C), lambda i: (i, 0))],
        out_shape=[jax.ShapeDtypeStruct((N, C), jnp.float32),
                   jax.ShapeDtypeStruct((N, C), jnp.float32)],
    )(features, w0, b0, w1, b1)


_BMC = 200

def _kc(gx0_ref, gx1_ref, nx_ref, nxyz_ref, sg0_ref, mg0_ref, sg1_ref, mg1_ref):
    pid = pl.program_id(0)

    @pl.when(pid == 0)
    def _():
        sg0_ref[...] = jnp.zeros_like(sg0_ref)
        mg0_ref[...] = jnp.zeros_like(mg0_ref)
        sg1_ref[...] = jnp.zeros_like(sg1_ref)
        mg1_ref[...] = jnp.zeros_like(mg1_ref)

    rows = pid * _BMC + lax.broadcasted_iota(jnp.int32, (_BMC, 1, 1), 0)
    mask = (rows % 64) == 0
    nx = nx_ref[...][:, None, :]
    g0 = jnp.where(mask, 0.0, gx0_ref[...] - nx)
    g1 = jnp.where(mask, 0.0, gx1_ref[...] - nx)
    nxyz_ref[...] = jnp.concatenate([g0[:, :, :3], g1[:, :, :3]], axis=1)
    gf0 = g0.reshape(_BMC * NS0, 16)
    gf1 = g1.reshape(_BMC * NS1, 16)
    sg0_ref[...] += lax.dot_general(gf0, gf0, (((0,), (0,)), ((), ())),
                                    preferred_element_type=jnp.float32, precision=lax.Precision.HIGHEST)
    mg0_ref[...] += jnp.sum(gf0, axis=0, keepdims=True)
    sg1_ref[...] += lax.dot_general(gf1, gf1, (((0,), (0,)), ((), ())),
                                    preferred_element_type=jnp.float32, precision=lax.Precision.HIGHEST)
    mg1_ref[...] += jnp.sum(gf1, axis=0, keepdims=True)


def _pos_pass(gx0, gx1, nxp):
    return pl.pallas_call(
        _kc,
        grid=(M // _BMC,),
        in_specs=[pl.BlockSpec((_BMC, NS0, 16), lambda i: (i, 0, 0)),
                  pl.BlockSpec((_BMC, NS1, 16), lambda i: (i, 0, 0)),
                  pl.BlockSpec((_BMC, 16), lambda i: (i, 0))],
        out_specs=[pl.BlockSpec((_BMC, NS0 + NS1, 3), lambda i: (i, 0, 0)),
                   pl.BlockSpec((16, 16), lambda i: (0, 0)),
                   pl.BlockSpec((1, 16), lambda i: (0, 0)),
                   pl.BlockSpec((16, 16), lambda i: (0, 0)),
                   pl.BlockSpec((1, 16), lambda i: (0, 0))],
        out_shape=[jax.ShapeDtypeStruct((M, NS0 + NS1, 3), jnp.float32),
                   jax.ShapeDtypeStruct((16, 16), jnp.float32),
                   jax.ShapeDtypeStruct((1, 16), jnp.float32),
                   jax.ShapeDtypeStruct((16, 16), jnp.float32),
                   jax.ShapeDtypeStruct((1, 16), jnp.float32)],
    )(gx0, gx1, nxp)


def _kd(gx0_ref, gx1_ref, f0_ref, f1_ref, nx_ref,
        wp0_ref, bp0_ref, wp1_ref, bp1_ref,
        nf_ref, pooled_ref, sp_ref, mp_ref):
    pid = pl.program_id(0)

    @pl.when(pid == 0)
    def _():
        sp_ref[...] = jnp.zeros_like(sp_ref)
        mp_ref[...] = jnp.zeros_like(mp_ref)

    rows = pid * _BMC + lax.broadcasted_iota(jnp.int32, (_BMC, 1, 1), 0)
    mask = (rows % 64) == 0
    nx = nx_ref[...][:, None, :]

    g0 = jnp.where(mask, 0.0, gx0_ref[...] - nx)
    g1 = jnp.where(mask, 0.0, gx1_ref[...] - nx)
    z0 = (jnp.dot(g0.reshape(_BMC * NS0, 16), wp0_ref[...],
                  preferred_element_type=jnp.float32, precision=lax.Precision.HIGHEST) + bp0_ref[...]).reshape(_BMC, NS0, C)
    z1 = (jnp.dot(g1.reshape(_BMC * NS1, 16), wp1_ref[...],
                  preferred_element_type=jnp.float32, precision=lax.Precision.HIGHEST) + bp1_ref[...]).reshape(_BMC, NS1, C)
    f0 = jnp.where(mask, 0.0, f0_ref[...])
    f1 = jnp.where(mask, 0.0, f1_ref[...])
    n0 = jnp.maximum(f0 + z0, 0.0)
    n1 = jnp.maximum(f1 + z1, 0.0)
    nf_ref[...] = jnp.concatenate([n0, n1], axis=1)
    pooled = jnp.concatenate([jnp.max(n0, axis=1), jnp.max(n1, axis=1)], axis=1)
    pooled_ref[...] = pooled
    sp_ref[...] += lax.dot_general(pooled, pooled, (((0,), (0,)), ((), ())),
                                   preferred_element_type=jnp.float32, precision=lax.Precision.HIGHEST)
    mp_ref[...] += jnp.sum(pooled, axis=0, keepdims=True)


def _main_pass(gx0, gx1, f0, f1, nxp, wp0, bp0, wp1, bp1):
    return pl.pallas_call(
        _kd,
        grid=(M // _BMC,),
        in_specs=[pl.BlockSpec((_BMC, NS0, 16), lambda i: (i, 0, 0)),
                  pl.BlockSpec((_BMC, NS1, 16), lambda i: (i, 0, 0)),
                  pl.BlockSpec((_BMC, NS0, C), lambda i: (i, 0, 0)),
                  pl.BlockSpec((_BMC, NS1, C), lambda i: (i, 0, 0)),
                  pl.BlockSpec((_BMC, 16), lambda i: (i, 0)),
                  pl.BlockSpec((16, C), lambda i: (0, 0)),
                  pl.BlockSpec((1, C), lambda i: (0, 0)),
                  pl.BlockSpec((16, C), lambda i: (0, 0)),
                  pl.BlockSpec((1, C), lambda i: (0, 0))],
        out_specs=[pl.BlockSpec((_BMC, NS0 + NS1, C), lambda i: (i, 0, 0)),
                   pl.BlockSpec((_BMC, 2 * C), lambda i: (i, 0)),
                   pl.BlockSpec((2 * C, 2 * C), lambda i: (0, 0)),
                   pl.BlockSpec((1, 2 * C), lambda i: (0, 0))],
        out_shape=[jax.ShapeDtypeStruct((M, NS0 + NS1, C), jnp.float32),
                   jax.ShapeDtypeStruct((M, 2 * C), jnp.float32),
                   jax.ShapeDtypeStruct((2 * C, 2 * C), jnp.float32),
                   jax.ShapeDtypeStruct((1, 2 * C), jnp.float32)],
    )(gx0, gx1, f0, f1, nxp, wp0, bp0, wp1, bp1)


_BME = 1000

def _ke(p_ref, w_ref, b_ref, o_ref):
    o_ref[...] = jnp.maximum(
        jnp.dot(p_ref[...], w_ref[...], preferred_element_type=jnp.float32, precision=lax.Precision.HIGHEST)
        + b_ref[...], 0.0)


def _out_pass(pooled, wb, bb):
    return pl.pallas_call(
        _ke,
        grid=(M // _BME,),
        in_specs=[pl.BlockSpec((_BME, 2 * C), lambda i: (i, 0)),
                  pl.BlockSpec((2 * C, 2 * C), lambda i: (0, 0)),
                  pl.BlockSpec((1, 2 * C), lambda i: (0, 0))],
        out_specs=pl.BlockSpec((_BME, 2 * C), lambda i: (i, 0)),
        out_shape=jax.ShapeDtypeStruct((M, 2 * C), jnp.float32),
    )(pooled, wb, bb)


def _fold_bn(S, m, W, cnt):
    mu = m / cnt
    mean_y = jnp.dot(mu, W, precision=lax.Precision.HIGHEST)
    ey2 = jnp.einsum('ac,ab,bc->c', W, S, W, precision=lax.Precision.HIGHEST) / cnt
    var = ey2 - mean_y[0] ** 2
    rs = lax.rsqrt(var + EPS)
    return W * rs[None, :], -mean_y * rs[None, :]


def kernel(xyz, xyz_batch_cnt, new_xyz, new_xyz_batch_cnt, new_coords, features,
           voxel2point_indices, W_in0, W_in1, W_pos0, W_pos1, W_out0, W_out1):
    del xyz_batch_cnt, new_xyz_batch_cnt, new_coords, voxel2point_indices

    S, m = _feat_moments(features)
    w0, b0 = _fold_bn(S, m, W_in0, float(N))
    w1, b1 = _fold_bn(S, m, W_in1, float(N))
    y0, y1 = _apply_in(features, w0, b0, w1, b1)

    txyz = jnp.pad(xyz, ((0, 0), (0, 13)))
    i0 = jnp.asarray(_IDX0P)
    i1 = jnp.asarray(_IDX1P)
    gf0r, gf1r, gx0r, gx1r = _sc_gather(y0, y1, txyz, i0, i1)
    gf0 = gf0r[:_B0].reshape(M, NS0, C)
    gf1 = gf1r[:_B1].reshape(M, NS1, C)
    gx0 = gx0r[:_B0].reshape(M, NS0, 16)
    gx1 = gx1r[:_B1].reshape(M, NS1, 16)

    nxp = jnp.pad(new_xyz, ((0, 0), (0, 13)))
    neighbor_xyz, sg0, mg0, sg1, mg1 = _pos_pass(gx0, gx1, nxp)

    wp0f = jnp.pad(W_pos0, ((0, 13), (0, 0)))
    wp1f = jnp.pad(W_pos1, ((0, 13), (0, 0)))
    wp0, bp0 = _fold_bn(sg0, mg0, wp0f, float(M * NS0))
    wp1, bp1 = _fold_bn(sg1, mg1, wp1f, float(M * NS1))

    neighbor_features, pooled, sp, mp = _main_pass(
        gx0, gx1, gf0, gf1, nxp, wp0, bp0, wp1, bp1)

    wo0, bo0 = _fold_bn(sp[:C, :C], mp[:, :C], W_out0, float(M))
    wo1, bo1 = _fold_bn(sp[C:, C:], mp[:, C:], W_out1, float(M))
    wb = jnp.zeros((2 * C, 2 * C), jnp.float32)
    wb = wb.at[:C, :C].set(wo0).at[C:, C:].set(wo1)
    bb = jnp.concatenate([bo0, bo1], axis=1)
    new_features = _out_pass(pooled, wb, bb)

    return new_features, neighbor_features, neighbor_xyz
